# Initial kernel scaffold; baseline (speedup 1.0000x reference)
import jax
import jax.numpy as jnp
from jax.experimental import pallas as pl


def kernel(x, edge_index, W1, b1, Wfc, bfc):
    raise NotImplementedError("write your pallas kernel here")


# speedup vs baseline: 34.7073x; 34.7073x over previous
import jax
import jax.numpy as jnp
from jax import lax
from jax.experimental import pallas as pl
from jax.experimental.pallas import tpu as pltpu
from jax.experimental.pallas import tpu_sc as plsc

F32 = jnp.float32

NC = 2
NS = 16
NW = NC * NS

N = 10000
NP = 10240
E = 320000
F = 128
EPW = E // NW
EPT = E // NS
CH = 80
FH = F // 2
RPT = NP // NS
BLK = 5120

_mesh = plsc.VectorSubcoreMesh(
    core_axis_name="c", subcore_axis_name="s", num_cores=NC, num_subcores=NS
)


def _worker_id():
    return lax.axis_index("s") * NC + lax.axis_index("c")


def _deg_body(ei_hbm, out_hbm, colv, hist, tmpv, redv, acc):
    cid = lax.axis_index("c")
    sid = lax.axis_index("s")
    wid = _worker_id()
    pltpu.sync_copy(ei_hbm.at[1, pl.ds(wid * EPW, EPW)], colv)
    zero16 = jnp.zeros((16,), F32)
    one16 = jnp.ones((16,), F32)

    def zbody(i, c):
        hist[pl.ds(i * 16, 16)] = zero16
        return c

    lax.fori_loop(0, NP // 16, zbody, 0)

    def body(k, c):
        idx = colv[pl.ds(k * 16, 16)]
        plsc.addupdate_scatter(hist, [idx], one16)
        return c

    lax.fori_loop(0, EPW // 16, body, 0)
    pltpu.sync_copy(hist, acc.at[sid])
    plsc.subcore_barrier()
    pltpu.sync_copy(acc.at[:, pl.ds(sid * RPT, RPT)], tmpv)

    def rbody(i, c):
        v = tmpv[0, pl.ds(i * 16, 16)]
        for t in range(1, NS):
            v = v + tmpv[t, pl.ds(i * 16, 16)]
        redv[pl.ds(i * 16, 16)] = v
        return c

    lax.fori_loop(0, RPT // 16, rbody, 0)
    pltpu.sync_copy(redv, out_hbm.at[cid, pl.ds(sid * RPT, RPT)])


def _deg_call(ei):
    return pl.kernel(
        _deg_body,
        out_type=jax.ShapeDtypeStruct((NC, NP), F32),
        mesh=_mesh,
        scratch_types=[
            pltpu.VMEM((EPW,), jnp.int32),
            pltpu.VMEM((NP,), F32),
            pltpu.VMEM((NS, RPT), F32),
            pltpu.VMEM((RPT,), F32),
            pltpu.VMEM_SHARED((NS, NP), F32),
        ],
        compiler_params=pltpu.CompilerParams(
            needs_layout_passes=False, use_tc_tiling_on_sc=False
        ),
    )(ei)


def _tc1_body(x_ref, w_ref, deg_ref, y_ref, y2_ref, dinv_ref):
    deg = 1.0 + deg_ref[0] + deg_ref[1]
    dinv = lax.rsqrt(deg)
    xw = jnp.dot(x_ref[...], w_ref[...], preferred_element_type=F32)
    y = xw * dinv[:, None]
    y_ref[...] = y
    y2_ref[0] = y[:, :FH]
    y2_ref[1] = y[:, FH:]
    dinv_ref[...] = dinv[:, None]


def _tc1_call(x, W1, deg2):
    blk = BLK
    return pl.pallas_call(
        _tc1_body,
        grid=(NP // blk,),
        in_specs=[
            pl.BlockSpec((blk, F), lambda i: (i, 0)),
            pl.BlockSpec((F, F), lambda i: (0, 0)),
            pl.BlockSpec((NC, blk), lambda i: (0, i)),
        ],
        out_specs=[
            pl.BlockSpec((blk, F), lambda i: (i, 0)),
            pl.BlockSpec((NC, blk, FH), lambda i: (0, i, 0)),
            pl.BlockSpec((blk, 1), lambda i: (i, 0)),
        ],
        out_shape=[
            jax.ShapeDtypeStruct((NP, F), F32),
            jax.ShapeDtypeStruct((NC, NP, FH), F32),
            jax.ShapeDtypeStruct((NP, 1), F32),
        ],
    )(x, W1, deg2)


def _scat_body(ei_hbm, y2_hbm, zeros_hbm, out_hbm, rowv, colv, b0, b1_, acc,
               sem0, sem1, ssem0, ssem1):
    cid = lax.axis_index("c")
    sid = lax.axis_index("s")
    z = pltpu.async_copy(zeros_hbm, acc.at[pl.ds(sid * RPT, RPT)], ssem0)
    r = pltpu.async_copy(ei_hbm.at[0, pl.ds(sid * EPT, EPT)], rowv, sem0)
    c = pltpu.async_copy(ei_hbm.at[1, pl.ds(sid * EPT, EPT)], colv, sem1)
    z.wait()
    r.wait()
    c.wait()
    plsc.subcore_barrier()
    ysrc = y2_hbm.at[cid]

    def gwait(j, buf, sem):
        pltpu.make_async_copy(ysrc.at[rowv.at[pl.ds(j, CH)]], buf, sem).wait()

    def swait(j, buf, sem):
        pltpu.make_async_copy(buf, acc.at[colv.at[pl.ds(j, CH)]], sem).wait()

    pltpu.async_copy(ysrc.at[rowv.at[pl.ds(0, CH)]], b0, sem0)
    pltpu.async_copy(ysrc.at[rowv.at[pl.ds(CH, CH)]], b1_, sem1)

    def body(t, carry):
        j0 = t * 2 * CH
        j1 = j0 + CH
        gwait(j0, b0, sem0)
        pltpu.async_copy(b0, acc.at[colv.at[pl.ds(j0, CH)]], ssem0, add=True)
        gwait(j1, b1_, sem1)
        pltpu.async_copy(b1_, acc.at[colv.at[pl.ds(j1, CH)]], ssem1, add=True)
        swait(j0, b0, ssem0)
        pltpu.async_copy(ysrc.at[rowv.at[pl.ds(j0 + 2 * CH, CH)]], b0, sem0)
        swait(j1, b1_, ssem1)
        pltpu.async_copy(ysrc.at[rowv.at[pl.ds(j1 + 2 * CH, CH)]], b1_, sem1)
        return carry

    lax.fori_loop(0, EPT // (2 * CH) - 1, body, 0)
    jt0 = EPT - 2 * CH
    jt1 = EPT - CH
    gwait(jt0, b0, sem0)
    pltpu.async_copy(b0, acc.at[colv.at[pl.ds(jt0, CH)]], ssem0, add=True)
    gwait(jt1, b1_, sem1)
    pltpu.async_copy(b1_, acc.at[colv.at[pl.ds(jt1, CH)]], ssem1, add=True)
    swait(jt0, b0, ssem0)
    swait(jt1, b1_, ssem1)
    plsc.subcore_barrier()
    pltpu.sync_copy(
        acc.at[pl.ds(sid * RPT, RPT)], out_hbm.at[cid, pl.ds(sid * RPT, RPT)]
    )


def _scat_call(ei, y2, zeros):
    return pl.kernel(
        _scat_body,
        out_type=jax.ShapeDtypeStruct((NC, NP, FH), F32),
        mesh=_mesh,
        scratch_types=[
            pltpu.VMEM((EPT,), jnp.int32),
            pltpu.VMEM((EPT,), jnp.int32),
            pltpu.VMEM((CH, FH), F32),
            pltpu.VMEM((CH, FH), F32),
            pltpu.VMEM_SHARED((NP, FH), F32),
            pltpu.SemaphoreType.DMA,
            pltpu.SemaphoreType.DMA,
            pltpu.SemaphoreType.DMA,
            pltpu.SemaphoreType.DMA,
        ],
        compiler_params=pltpu.CompilerParams(
            needs_layout_passes=False, use_tc_tiling_on_sc=False
        ),
    )(ei, y2, zeros)


def _tc2_body(s_ref, y_ref, dinv_ref, b1_ref, wfc_ref, bfc_ref, ab_ref):
    s = jnp.concatenate([s_ref[0], s_ref[1]], axis=1)
    h = dinv_ref[...] * (s + y_ref[...]) + b1_ref[...]
    wd = wfc_ref[:, 0] - wfc_ref[:, 1]
    wd2 = jnp.concatenate([wd[:F, None], wd[F:, None]], axis=1)
    ab = jnp.dot(h, wd2, preferred_element_type=F32)
    c0 = bfc_ref[0, 0] - bfc_ref[0, 1]
    ab_ref[...] = jnp.concatenate([ab[:, 0:1] + c0, ab[:, 1:2]], axis=1).T


def _tc2_call(s2, y, dinv, b1, Wfc, bfc):
    blk = BLK
    return pl.pallas_call(
        _tc2_body,
        grid=(NP // blk,),
        in_specs=[
            pl.BlockSpec((NC, blk, FH), lambda i: (0, i, 0)),
            pl.BlockSpec((blk, F), lambda i: (i, 0)),
            pl.BlockSpec((blk, 1), lambda i: (i, 0)),
            pl.BlockSpec((1, F), lambda i: (0, 0)),
            pl.BlockSpec((2 * F, 2), lambda i: (0, 0)),
            pl.BlockSpec((1, 2), lambda i: (0, 0)),
        ],
        out_specs=pl.BlockSpec((2, blk), lambda i: (0, i)),
        out_shape=jax.ShapeDtypeStruct((2, NP), F32),
    )(s2, y, dinv, b1.reshape(1, F), Wfc, bfc.reshape(1, 2))


def _edge_body(ei_hbm, ab_hbm, out_hbm, rowv, colv, av, bv, p0v, p1v,
               esem0, esem1, esem2, esem3):
    wid = _worker_id()
    base = wid * EPW
    r = pltpu.async_copy(ei_hbm.at[0, pl.ds(base, EPW)], rowv, esem0)
    c = pltpu.async_copy(ei_hbm.at[1, pl.ds(base, EPW)], colv, esem1)
    a = pltpu.async_copy(ab_hbm.at[0], av, esem2)
    b = pltpu.async_copy(ab_hbm.at[1], bv, esem3)
    r.wait()
    c.wait()
    a.wait()
    b.wait()

    @plsc.parallel_loop(0, EPW, 16, unroll=8)
    def body(k):
        r = rowv[pl.ds(k, 16)]
        c = colv[pl.ds(k, 16)]
        va = plsc.load_gather(av, [c])
        vb = plsc.load_gather(bv, [r])
        d = va + vb
        e = jnp.exp(-d)
        p0 = 1.0 / (1.0 + e)
        p0v[pl.ds(k, 16)] = p0
        p1v[pl.ds(k, 16)] = 1.0 - p0
    pltpu.sync_copy(p0v, out_hbm.at[0, pl.ds(base, EPW)])
    pltpu.sync_copy(p1v, out_hbm.at[1, pl.ds(base, EPW)])


def _edge_call(ei, ab):
    return pl.kernel(
        _edge_body,
        out_type=jax.ShapeDtypeStruct((2, E), F32),
        mesh=_mesh,
        scratch_types=[
            pltpu.VMEM((EPW,), jnp.int32),
            pltpu.VMEM((EPW,), jnp.int32),
            pltpu.VMEM((NP,), F32),
            pltpu.VMEM((NP,), F32),
            pltpu.VMEM((EPW,), F32),
            pltpu.VMEM((EPW,), F32),
            pltpu.SemaphoreType.DMA,
            pltpu.SemaphoreType.DMA,
            pltpu.SemaphoreType.DMA,
            pltpu.SemaphoreType.DMA,
        ],
        compiler_params=pltpu.CompilerParams(
            needs_layout_passes=False, use_tc_tiling_on_sc=False
        ),
    )(ei, ab)


@jax.jit
def kernel(x, edge_index, W1, b1, Wfc, bfc):
    zeros_acc = jnp.zeros((RPT, FH), F32)

    x_pad = jnp.pad(x, ((0, NP - N), (0, 0)))
    deg2 = _deg_call(edge_index)
    y, y2, dinv = _tc1_call(x_pad, W1, deg2)
    s2 = _scat_call(edge_index, y2, zeros_acc)
    ab = _tc2_call(s2, y, dinv, b1, Wfc, bfc)
    return _edge_call(edge_index, ab).T
